# baseline (speedup 1.0000x reference)
import functools

import jax
import jax.numpy as jnp
from jax import lax
from jax.experimental import pallas as pl
from jax.experimental.pallas import tpu as pltpu
from jax.experimental.pallas import tpu_sc as plsc


@functools.cache
def _build_gather(B, V, D, nc, ns):
    nw = nc * ns
    b_per_w = B // nw
    ch = 128
    n_chunks = b_per_w // ch

    mesh = plsc.VectorSubcoreMesh(core_axis_name="c", subcore_axis_name="s")

    @functools.partial(
        pl.kernel,
        mesh=mesh,
        out_type=jax.ShapeDtypeStruct((B, D), jnp.float32),
        scratch_types=[
            pltpu.VMEM((b_per_w,), jnp.int32),
            pltpu.VMEM((b_per_w,), jnp.int32),
            pltpu.VMEM((ch, 2 * D), jnp.float32),
            pltpu.VMEM((ch, 2 * D), jnp.float32),
            pltpu.VMEM((ch, D), jnp.float32),
            pltpu.SemaphoreType.DMA,
            pltpu.SemaphoreType.DMA,
        ],
    )
    def gather(idx_hbm, table_hbm, out_hbm, blk_v, sel_v,
               rows0, rows1, outb, sem0, sem1):
        wid = lax.axis_index("s") * nc + lax.axis_index("c")
        base = wid * b_per_w
        pltpu.sync_copy(idx_hbm.at[pl.ds(base, b_per_w)], blk_v)
        for t in range(b_per_w // 16):
            v = blk_v[pl.ds(t * 16, 16)]
            sel_v[pl.ds(t * 16, 16)] = lax.bitwise_and(v, 1)
            blk_v[pl.ds(t * 16, 16)] = lax.shift_right_logical(v, 1)

        rows = (rows0, rows1)
        sems = (sem0, sem1)
        copies = [None, None]
        copies[0] = pltpu.async_copy(
            table_hbm.at[blk_v.at[pl.ds(0, ch)]], rows[0], sems[0])
        for j in range(n_chunks):
            b = j & 1
            nb = b ^ 1
            if j + 1 < n_chunks:
                copies[nb] = pltpu.async_copy(
                    table_hbm.at[blk_v.at[pl.ds((j + 1) * ch, ch)]],
                    rows[nb], sems[nb])
            copies[b].wait()
            rows_b = rows[b]

            def select_group(g, carry, j=j, rows_b=rows_b):
                p16 = sel_v[pl.ds(j * ch + g * 16, 16)]
                for lane in range(16):
                    n = g * 16 + lane
                    w = jnp.full((16,), p16[lane].astype(jnp.float32),
                                 jnp.float32)
                    for c in range(D // 16):
                        lo = rows_b[n, pl.ds(c * 16, 16)]
                        hi = rows_b[n, pl.ds(D + c * 16, 16)]
                        outb[n, pl.ds(c * 16, 16)] = lo + (hi - lo) * w
                return carry

            lax.fori_loop(0, ch // 16, select_group, 0)
            pltpu.sync_copy(outb, out_hbm.at[pl.ds(base + j * ch, ch)])

    return gather


def kernel(entity_ids, entity_embedding):
    (B,) = entity_ids.shape
    V, D = entity_embedding.shape
    info = plsc.get_sparse_core_info()
    gather = _build_gather(B, V, D, info.num_cores, info.num_subcores)
    table2 = entity_embedding.reshape(V // 2, 2 * D)
    return gather(entity_ids.astype(jnp.int32), table2)


# speedup vs baseline: 2.3519x; 2.3519x over previous
import functools

import jax
import jax.numpy as jnp
from jax import lax
from jax.experimental import pallas as pl
from jax.experimental.pallas import tpu as pltpu
from jax.experimental.pallas import tpu_sc as plsc


@functools.cache
def _build_gather(B, V, D, nc, ns):
    nw = nc * ns
    b_per_w = B // nw
    ch = 32
    n_pairs = b_per_w // (2 * ch)

    mesh = plsc.VectorSubcoreMesh(core_axis_name="c", subcore_axis_name="s")

    @functools.partial(
        pl.kernel,
        mesh=mesh,
        out_type=jax.ShapeDtypeStruct((B, D), jnp.float32),
        scratch_types=[
            pltpu.VMEM((b_per_w,), jnp.int32),
            pltpu.VMEM((b_per_w,), jnp.int32),
            pltpu.VMEM((ch * 8, D), jnp.float32),
            pltpu.VMEM((ch * 8, D), jnp.float32),
            pltpu.VMEM((ch, D), jnp.float32),
            pltpu.VMEM((ch, D), jnp.float32),
            pltpu.SemaphoreType.DMA,
            pltpu.SemaphoreType.DMA,
        ],
    )
    def gather(idx_hbm, table_hbm, out_hbm, blk_v, row_v,
               rows0, rows1, outb0, outb1, sem0, sem1):
        wid = lax.axis_index("s") * nc + lax.axis_index("c")
        base = wid * b_per_w
        pltpu.sync_copy(idx_hbm.at[pl.ds(base, b_per_w)], blk_v)
        for t in range(b_per_w // 16):
            v = blk_v[pl.ds(t * 16, 16)]
            row_v[pl.ds(t * 16, 16)] = lax.bitwise_and(v, 7)
            blk_v[pl.ds(t * 16, 16)] = lax.shift_right_logical(v, 3)

        rows = (rows0, rows1)
        outb = (outb0, outb1)
        sems = (sem0, sem1)

        def fire(j, b):
            for g in range(ch // 16):
                v16 = blk_v[pl.ds(j * ch + g * 16, 16)]
                for lane in range(16):
                    n = g * 16 + lane
                    pltpu.async_copy(
                        table_hbm.at[v16[lane]],
                        rows[b].at[pl.ds(n * 8, 8)], sems[b])

        def process(j, b):
            pltpu.make_async_copy(
                out_hbm.at[pl.ds(0, ch * 8)], rows[b], sems[b]).wait()

            def select_group(g, carry, j=j, b=b):
                p16 = row_v[pl.ds(j * ch + g * 16, 16)]
                for lane in range(16):
                    n = g * 16 + lane
                    p = p16[lane]
                    for c in range(D // 16):
                        outb[b][n, pl.ds(c * 16, 16)] = (
                            rows[b][n * 8 + p, pl.ds(c * 16, 16)])
                return carry

            lax.fori_loop(0, ch // 16, select_group, 0)
            pltpu.sync_copy(outb[b], out_hbm.at[pl.ds(base + j * ch, ch)])

        def pair_body(m, carry):
            process(2 * m, 0)
            fire(2 * m + 2, 0)
            process(2 * m + 1, 1)
            fire(2 * m + 3, 1)
            return carry

        fire(0, 0)
        fire(1, 1)
        lax.fori_loop(0, n_pairs - 1, pair_body, 0)
        process(2 * (n_pairs - 1), 0)
        process(2 * (n_pairs - 1) + 1, 1)

    return gather


def kernel(entity_ids, entity_embedding):
    (B,) = entity_ids.shape
    V, D = entity_embedding.shape
    info = plsc.get_sparse_core_info()
    gather = _build_gather(B, V, D, info.num_cores, info.num_subcores)
    table3 = entity_embedding.reshape(V // 8, 8, D)
    return gather(entity_ids.astype(jnp.int32), table3)


# speedup vs baseline: 2.3771x; 1.0107x over previous
import functools

import jax
import jax.numpy as jnp
from jax import lax
from jax.experimental import pallas as pl
from jax.experimental.pallas import tpu as pltpu
from jax.experimental.pallas import tpu_sc as plsc


@functools.cache
def _build_gather(B, V, D, nc, ns):
    nw = nc * ns
    b_per_w = B // nw
    ch = 16
    nbuf = 4
    n_chunks = b_per_w // ch
    n_loop = n_chunks // nbuf - 1

    mesh = plsc.VectorSubcoreMesh(core_axis_name="c", subcore_axis_name="s")

    @functools.partial(
        pl.kernel,
        mesh=mesh,
        out_type=jax.ShapeDtypeStruct((B, D), jnp.float32),
        scratch_types=[
            pltpu.VMEM((b_per_w,), jnp.int32),
            pltpu.VMEM((b_per_w,), jnp.int32),
            pltpu.VMEM((ch * 8, D), jnp.float32),
            pltpu.VMEM((ch * 8, D), jnp.float32),
            pltpu.VMEM((ch * 8, D), jnp.float32),
            pltpu.VMEM((ch * 8, D), jnp.float32),
            pltpu.VMEM((ch, D), jnp.float32),
            pltpu.SemaphoreType.DMA,
            pltpu.SemaphoreType.DMA,
            pltpu.SemaphoreType.DMA,
            pltpu.SemaphoreType.DMA,
        ],
    )
    def gather(idx_hbm, table_hbm, out_hbm, blk_v, row_v,
               rows0, rows1, rows2, rows3, outb,
               sem0, sem1, sem2, sem3):
        wid = lax.axis_index("s") * nc + lax.axis_index("c")
        base = wid * b_per_w
        pltpu.sync_copy(idx_hbm.at[pl.ds(base, b_per_w)], blk_v)
        for t in range(b_per_w // 16):
            v = blk_v[pl.ds(t * 16, 16)]
            row_v[pl.ds(t * 16, 16)] = lax.bitwise_and(v, 7)
            blk_v[pl.ds(t * 16, 16)] = lax.shift_right_logical(v, 3)

        rows = (rows0, rows1, rows2, rows3)
        sems = (sem0, sem1, sem2, sem3)

        def fire(j, b):
            for g in range(ch // 16):
                v16 = blk_v[pl.ds(j * ch + g * 16, 16)]
                for lane in range(16):
                    n = g * 16 + lane
                    pltpu.async_copy(
                        table_hbm.at[v16[lane]],
                        rows[b].at[pl.ds(n * 8, 8)], sems[b])

        def process(j, b):
            pltpu.make_async_copy(
                out_hbm.at[pl.ds(0, ch * 8)], rows[b], sems[b]).wait()

            def select_group(g, carry, j=j, b=b):
                p16 = row_v[pl.ds(j * ch + g * 16, 16)]
                for lane in range(16):
                    n = g * 16 + lane
                    p = p16[lane]
                    for c in range(D // 16):
                        outb[n, pl.ds(c * 16, 16)] = (
                            rows[b][n * 8 + p, pl.ds(c * 16, 16)])
                return carry

            lax.fori_loop(0, ch // 16, select_group, 0)
            pltpu.sync_copy(outb, out_hbm.at[pl.ds(base + j * ch, ch)])

        for b in range(nbuf):
            fire(b, b)

        def loop_body(m, carry):
            for b in range(nbuf):
                j = nbuf * m + b
                process(j, b)
                fire(j + nbuf, b)
            return carry

        lax.fori_loop(0, n_loop, loop_body, 0)
        for b in range(nbuf):
            process(nbuf * n_loop + b, b)

    return gather


def kernel(entity_ids, entity_embedding):
    (B,) = entity_ids.shape
    V, D = entity_embedding.shape
    info = plsc.get_sparse_core_info()
    gather = _build_gather(B, V, D, info.num_cores, info.num_subcores)
    table3 = entity_embedding.reshape(V // 8, 8, D)
    return gather(entity_ids.astype(jnp.int32), table3)


# speedup vs baseline: 2.3817x; 1.0019x over previous
import functools

import jax
import jax.numpy as jnp
from jax import lax
from jax.experimental import pallas as pl
from jax.experimental.pallas import tpu as pltpu
from jax.experimental.pallas import tpu_sc as plsc


@functools.cache
def _build_gather(B, V, D, nc, ns):
    nw = nc * ns
    b_per_w = B // nw
    ch = 16
    nbuf = 4
    n_chunks = b_per_w // ch
    n_loop = n_chunks // nbuf - 1

    mesh = plsc.VectorSubcoreMesh(core_axis_name="c", subcore_axis_name="s")

    @functools.partial(
        pl.kernel,
        mesh=mesh,
        out_type=jax.ShapeDtypeStruct((B, D), jnp.float32),
        scratch_types=[
            pltpu.VMEM((b_per_w,), jnp.int32),
            pltpu.VMEM((b_per_w,), jnp.int32),
            pltpu.VMEM((ch * 8, D), jnp.float32),
            pltpu.VMEM((ch * 8, D), jnp.float32),
            pltpu.VMEM((ch * 8, D), jnp.float32),
            pltpu.VMEM((ch * 8, D), jnp.float32),
            pltpu.VMEM((nbuf * ch, D), jnp.float32),
            pltpu.SemaphoreType.DMA,
            pltpu.SemaphoreType.DMA,
            pltpu.SemaphoreType.DMA,
            pltpu.SemaphoreType.DMA,
        ],
    )
    def gather(idx_hbm, table_hbm, out_hbm, blk_v, row_v,
               rows0, rows1, rows2, rows3, outb,
               sem0, sem1, sem2, sem3):
        wid = lax.axis_index("s") * nc + lax.axis_index("c")
        base = wid * b_per_w
        pltpu.sync_copy(idx_hbm.at[pl.ds(base, b_per_w)], blk_v)
        for t in range(b_per_w // 16):
            v = blk_v[pl.ds(t * 16, 16)]
            row_v[pl.ds(t * 16, 16)] = lax.bitwise_and(v, 7)
            blk_v[pl.ds(t * 16, 16)] = lax.shift_right_logical(v, 3)

        rows = (rows0, rows1, rows2, rows3)
        sems = (sem0, sem1, sem2, sem3)

        def fire(j, b):
            for g in range(ch // 16):
                v16 = blk_v[pl.ds(j * ch + g * 16, 16)]
                for lane in range(16):
                    n = g * 16 + lane
                    pltpu.async_copy(
                        table_hbm.at[v16[lane]],
                        rows[b].at[pl.ds(n * 8, 8)], sems[b])

        def process(j, b):
            pltpu.make_async_copy(
                out_hbm.at[pl.ds(0, ch * 8)], rows[b], sems[b]).wait()

            def select_group(g, carry, j=j, b=b):
                p16 = row_v[pl.ds(j * ch + g * 16, 16)]
                for lane in range(16):
                    n = g * 16 + lane
                    p = p16[lane]
                    for c in range(D // 16):
                        outb[b * ch + n, pl.ds(c * 16, 16)] = (
                            rows[b][n * 8 + p, pl.ds(c * 16, 16)])
                return carry

            lax.fori_loop(0, ch // 16, select_group, 0)

        for b in range(nbuf):
            fire(b, b)

        def loop_body(m, carry):
            for b in range(nbuf):
                j = nbuf * m + b
                process(j, b)
                fire(j + nbuf, b)
            pltpu.sync_copy(
                outb, out_hbm.at[pl.ds(base + m * nbuf * ch, nbuf * ch)])
            return carry

        lax.fori_loop(0, n_loop, loop_body, 0)
        for b in range(nbuf):
            process(nbuf * n_loop + b, b)
        pltpu.sync_copy(
            outb, out_hbm.at[pl.ds(base + n_loop * nbuf * ch, nbuf * ch)])

    return gather


def kernel(entity_ids, entity_embedding):
    (B,) = entity_ids.shape
    V, D = entity_embedding.shape
    info = plsc.get_sparse_core_info()
    gather = _build_gather(B, V, D, info.num_cores, info.num_subcores)
    table3 = entity_embedding.reshape(V // 8, 8, D)
    return gather(entity_ids.astype(jnp.int32), table3)
